# baseline (speedup 1.0000x reference)
import functools

import jax
import jax.numpy as jnp
import numpy as np
from jax import lax
from jax.experimental import pallas as pl
from jax.experimental.pallas import tpu as pltpu
from jax.experimental.pallas import tpu_sc as plsc

P_MASK = 0.5
TIME_MASKING_PARA = 100
TIME_MASK_NUM = 1

RPC = 200
NW = 32


def _mask_bounds(B, T):
    with jax.ensure_compile_time_eval():
        return _mask_bounds_traced(B, T)


def _mask_bounds_traced(B, T):
    key = jax.random.key(42)
    key, k_apply = jax.random.split(key)
    apply_mask = jax.random.uniform(k_apply) <= P_MASK
    starts_l, ends_l = [], []
    for _ in range(TIME_MASK_NUM):
        key, k_t, k_t0 = jax.random.split(key, 3)
        t = jax.random.randint(k_t, (), 0, TIME_MASKING_PARA + 1)
        t0s = jax.random.randint(k_t0, (B,), 0, T - TIME_MASKING_PARA)
        t_eff = jnp.where(apply_mask, t, 0)
        starts_l.append(t0s.astype(jnp.int32))
        ends_l.append((t0s + t_eff).astype(jnp.int32))
    return starts_l[0], ends_l[0]


def _make_sc_kernel(B, F, T, t_width, tp):
    nch = T // RPC
    total = 2 * nch
    mesh = plsc.VectorSubcoreMesh(core_axis_name="c", subcore_axis_name="s")

    scratch = [
        pltpu.VMEM((RPC, F), jnp.float32),
        pltpu.VMEM((RPC, F), jnp.float32),
        pltpu.SemaphoreType.DMA,
        pltpu.SemaphoreType.DMA,
        pltpu.SemaphoreType.DMA,
        pltpu.SemaphoreType.DMA,
    ]
    if t_width:
        scratch += [
            pltpu.VMEM((tp,), jnp.int32),
            pltpu.VMEM((tp, F), jnp.float32),
            pltpu.SemaphoreType.DMA,
            pltpu.SemaphoreType.DMA,
        ]

    @functools.partial(
        pl.kernel, mesh=mesh,
        out_type=jax.ShapeDtypeStruct((B * T, F), jnp.float32),
        scratch_types=scratch,
    )
    def k(x_hbm, idx_hbm, z_hbm, o_hbm, *rest):
        if t_width:
            (b0, b1, is0, is1, os0, os1, idxv, zv, isem, ssem) = rest
        else:
            (b0, b1, is0, is1, os0, os1) = rest
            idx_hbm, z_hbm = None, None
        bufs = (b0, b1)
        in_sems = (is0, is1)
        out_sems = (os0, os1)
        wid = lax.axis_index("s") * 2 + lax.axis_index("c")

        def base(g):
            b = wid * 2 + g // nch
            return b * T + (g % nch) * RPC

        def in_copy(g):
            p = g % 2
            return pltpu.make_async_copy(
                x_hbm.at[pl.ds(base(g), RPC), :], bufs[p], in_sems[p])

        def out_copy(g):
            p = g % 2
            return pltpu.make_async_copy(
                bufs[p], o_hbm.at[pl.ds(base(g), RPC), :], out_sems[p])

        in_copy(0).start()
        for g in range(total):
            in_copy(g).wait()
            out_copy(g).start()
            ng = g + 1
            if ng < total:
                if ng >= 2:
                    out_copy(ng - 2).wait()
                in_copy(ng).start()
        out_copy(total - 2).wait()
        out_copy(total - 1).wait()

        if t_width:
            pltpu.make_async_copy(
                idx_hbm.at[pl.ds(wid * tp, tp)], idxv, isem).start()
            pltpu.make_async_copy(
                idx_hbm.at[pl.ds(wid * tp, tp)], idxv, isem).wait()
            pltpu.make_async_copy(z_hbm, zv, ssem).start()
            pltpu.make_async_copy(z_hbm, zv, ssem).wait()
            pltpu.make_async_copy(zv, o_hbm.at[idxv], ssem).start()
            pltpu.make_async_copy(zv, o_hbm.at[idxv], ssem).wait()

    return k


def kernel(mel_batch):
    B, F, T = mel_batch.shape
    starts, ends = _mask_bounds(B, T)
    s_np = np.asarray(starts)
    e_np = np.asarray(ends)
    t_width = int(e_np[0] - s_np[0])
    tp = ((2 * t_width + 7) // 8) * 8 if t_width else 0

    xt = jnp.transpose(mel_batch, (0, 2, 1))
    x3 = xt.reshape(B * T, F)

    if t_width:
        idx = np.empty((NW, tp), dtype=np.int32)
        for w in range(NW):
            rows = []
            for b in (2 * w, 2 * w + 1):
                rows.extend(range(b * T + int(s_np[b]),
                                  b * T + int(e_np[b])))
            while len(rows) < tp:
                rows.append(rows[0])
            idx[w] = rows
        idx_arr = jnp.asarray(idx.reshape(-1))
        z_arr = jnp.zeros((tp, F), jnp.float32)
    else:
        idx_arr = jnp.zeros((8,), jnp.int32)
        z_arr = jnp.zeros((8, F), jnp.float32)

    sc = _make_sc_kernel(B, F, T, t_width, tp)
    out3 = sc(x3, idx_arr, z_arr)
    return jnp.transpose(out3.reshape(B, T, F), (0, 2, 1))


# speedup vs baseline: 1.5177x; 1.5177x over previous
import jax
import jax.numpy as jnp
from jax import lax
from jax.experimental import pallas as pl
from jax.experimental.pallas import tpu as pltpu

P_MASK = 0.5
TIME_MASKING_PARA = 100
TIME_MASK_NUM = 1

NBUF = 12
LAG = 4
WIN = 128


def _mask_bounds(B, T):
    with jax.ensure_compile_time_eval():
        return _mask_bounds_traced(B, T)


def _mask_bounds_traced(B, T):
    key = jax.random.key(42)
    key, k_apply = jax.random.split(key)
    apply_mask = jax.random.uniform(k_apply) <= P_MASK
    starts_l, ends_l = [], []
    for _ in range(TIME_MASK_NUM):
        key, k_t, k_t0 = jax.random.split(key, 3)
        t = jax.random.randint(k_t, (), 0, TIME_MASKING_PARA + 1)
        t0s = jax.random.randint(k_t0, (B,), 0, T - TIME_MASKING_PARA)
        t_eff = jnp.where(apply_mask, t, 0)
        starts_l.append(t0s.astype(jnp.int32))
        ends_l.append((t0s + t_eff).astype(jnp.int32))
    return starts_l[0], ends_l[0]


def _make_body(B, F, T):
    def body(starts_ref, ends_ref, w0s_ref, x_hbm, o_hbm, *rest):
        bufs = rest[0:NBUF]
        in_sems = rest[NBUF:2 * NBUF]
        out_sems = rest[2 * NBUF:3 * NBUF]

        def in_copy(i, slot):
            return pltpu.make_async_copy(
                x_hbm.at[i], bufs[slot], in_sems[slot])

        def out_copy(i, slot):
            return pltpu.make_async_copy(
                bufs[slot], o_hbm.at[i], out_sems[slot])

        riota = lax.broadcasted_iota(jnp.int32, (WIN, F), 0)
        for i in range(NBUF):
            in_copy(i, i).start(priority=i % 2)
        for i in range(B):
            slot = i % NBUF
            in_copy(i, slot).wait()
            s = starts_ref[i]
            e = ends_ref[i]
            w0 = w0s_ref[i]
            rows = riota + w0
            slab = bufs[slot][pl.ds(w0, WIN), :]
            zero = (rows >= s) & (rows < e)
            bufs[slot][pl.ds(w0, WIN), :] = jnp.where(
                zero, jnp.float32(0.0), slab)
            out_copy(i, slot).start(priority=slot % 2)
            j = i - LAG
            if j >= 0:
                out_copy(j, j % NBUF).wait()
                nxt = j + NBUF
                if nxt < B:
                    in_copy(nxt, j % NBUF).start(priority=nxt % 2)
        for j in range(B - LAG, B):
            out_copy(j, j % NBUF).wait()

    return body


def kernel(mel_batch):
    B, F, T = mel_batch.shape
    starts, ends = _mask_bounds(B, T)
    w0s = jnp.minimum((starts // 8) * 8, T - WIN)
    xt = jnp.transpose(mel_batch, (0, 2, 1))
    out_t = pl.pallas_call(
        _make_body(B, F, T),
        grid=(),
        in_specs=[
            pl.BlockSpec(memory_space=pltpu.SMEM),
            pl.BlockSpec(memory_space=pltpu.SMEM),
            pl.BlockSpec(memory_space=pltpu.SMEM),
            pl.BlockSpec(memory_space=pl.ANY),
        ],
        out_specs=pl.BlockSpec(memory_space=pl.ANY),
        out_shape=jax.ShapeDtypeStruct((B, T, F), jnp.float32),
        scratch_shapes=(
            [pltpu.VMEM((T, F), jnp.float32) for _ in range(NBUF)]
            + [pltpu.SemaphoreType.DMA for _ in range(2 * NBUF)]
        ),
    )(starts, ends, w0s, xt)
    return jnp.transpose(out_t, (0, 2, 1))
